# baseline (speedup 1.0000x reference)
import functools

import numpy as np
import jax
import jax.numpy as jnp
from jax import lax
from jax.experimental import pallas as pl
from jax.experimental.pallas import tpu as pltpu
from jax.experimental.pallas import tpu_sc as plsc

N = 64
NUM_WORKERS = 32
TOTAL_ROWS = 16 * 512
ROWS_PER_WORKER = TOTAL_ROWS // NUM_WORKERS


def _valid_mask_np():
    i = np.arange(N)[:, None]
    j = np.arange(N)[None, :]
    d = j - i
    m = (((d >= 0) & (d <= 15))
         | ((d >= 17) & (d <= 31) & (d % 2 == 1) & (i % 2 == 0))
         | ((d >= 35) & (d <= 63) & (d % 4 == 3) & (i % 4 == 0)))
    return m.astype(np.float32)


_MASK_NP = _valid_mask_np()
_WRITE = [[c for c in range(4) if _MASK_NP[i, c * 16:(c + 1) * 16].any()]
          for i in range(N)]

_GATHER_DNUMS = lax.GatherDimensionNumbers(
    offset_dims=(), collapsed_slice_dims=(0,), start_index_map=(0,))


def _splat(vec, r, lane):
    idx = ((lane & 0) + r)[:, None]
    return lax.gather(vec, idx, _GATHER_DNUMS, (1,),
                      mode=lax.GatherScatterMode.PROMISE_IN_BOUNDS)


NBUF = 4


def _sc_band_max(x_hbm, mask_hbm, out_hbm,
                 xbuf, maskbuf, tile0, tile1, tile2, tile3,
                 sem0, sem1, sem2, sem3):
    wid = lax.axis_index("s") * 2 + lax.axis_index("c")
    base = wid * ROWS_PER_WORKER
    pltpu.sync_copy(x_hbm.at[pl.ds(base, ROWS_PER_WORKER)], xbuf)
    pltpu.sync_copy(mask_hbm, maskbuf)

    lane = lax.iota(jnp.int32, 16)
    zero = (lane & 0).astype(jnp.float32)

    def zero_body(i, carry):
        for buf in (tile0, tile1, tile2, tile3):
            for c in range(4):
                buf[i, pl.ds(c * 16, 16)] = zero
        return carry

    lax.fori_loop(0, N, zero_body, 0)

    def compute_tile(t, tilebuf):
        xb = [xbuf[t, pl.ds(c * 16, 16)] for c in range(4)]
        M = [None] * 4
        for b in (3, 2, 1, 0):
            for r in range(15, -1, -1):
                i = 16 * b + r
                sp = _splat(xb[b], r, lane)
                for c in range(b, 4):
                    M[c] = sp if M[c] is None else jnp.maximum(sp, M[c])
                if r != 15:
                    M[b] = jnp.where(lane == r, sp, M[b])
                for c in _WRITE[i]:
                    mv = maskbuf[i, pl.ds(c * 16, 16)]
                    tilebuf[i, pl.ds(c * 16, 16)] = M[c] * mv

    bufs = ((tile0, sem0), (tile1, sem1), (tile2, sem2), (tile3, sem3))

    def ring_body(tg, carry):
        for phase, (buf, sem) in enumerate(bufs):
            t = NBUF * tg + phase
            cp = pltpu.make_async_copy(buf, out_hbm.at[base + t], sem)

            @pl.when(tg > 0)
            def _():
                cp.wait()

            compute_tile(t, buf)
            cp.start()
        return carry

    lax.fori_loop(0, ROWS_PER_WORKER // NBUF, ring_body, 0)
    for phase, (buf, sem) in enumerate(bufs):
        pltpu.make_async_copy(
            buf, out_hbm.at[base + ROWS_PER_WORKER - NBUF + phase], sem).wait()


@jax.jit
def _run(xf, mask):
    mesh = plsc.VectorSubcoreMesh(core_axis_name="c", subcore_axis_name="s")
    f = pl.kernel(
        _sc_band_max,
        mesh=mesh,
        out_type=jax.ShapeDtypeStruct((TOTAL_ROWS, N, N), jnp.float32),
        scratch_types=[
            pltpu.VMEM((ROWS_PER_WORKER, N), jnp.float32),
            pltpu.VMEM((N, N), jnp.float32),
            pltpu.VMEM((N, N), jnp.float32),
            pltpu.VMEM((N, N), jnp.float32),
            pltpu.VMEM((N, N), jnp.float32),
            pltpu.VMEM((N, N), jnp.float32),
            pltpu.SemaphoreType.DMA,
            pltpu.SemaphoreType.DMA,
            pltpu.SemaphoreType.DMA,
            pltpu.SemaphoreType.DMA,
        ],
    )
    return f(xf, mask)


def kernel(x):
    B, D, n = x.shape
    xf = x.reshape(B * D, n)
    mask = jnp.asarray(_MASK_NP)
    out = _run(xf, mask)
    return out.reshape(B, D, n, n)


# speedup vs baseline: 3.2036x; 3.2036x over previous
import functools

import numpy as np
import jax
import jax.numpy as jnp
from jax import lax
from jax.experimental import pallas as pl
from jax.experimental.pallas import tpu as pltpu
from jax.experimental.pallas import tpu_sc as plsc

N = 64
CH = 512
NL = CH // 16
B = 16


def _load_row(xbuf, j):
    return [xbuf[j, pl.ds(k * 16, 16)] for k in range(NL)]


def _absorb(M, xbuf, j):
    return [jnp.maximum(m, v) for m, v in zip(M, _load_row(xbuf, j))]


def _store_row(buf, j, vals):
    for k in range(NL):
        buf[j, pl.ds(k * 16, 16)] = vals[k]


def _do_row(i, xbuf, rowbuf, zeros):
    def zpre(jj, c):
        _store_row(rowbuf, jj, zeros)
        return c
    lax.fori_loop(0, i, zpre, 0)

    M = tuple(_load_row(xbuf, i))
    _store_row(rowbuf, i, M)

    trip_a = jnp.minimum(15, 63 - i)

    def abody(k, M):
        j = i + k
        Mn = _absorb(list(M), xbuf, j)
        _store_row(rowbuf, j, Mn)
        return tuple(Mn)
    M = lax.fori_loop(1, trip_a + 1, abody, M)

    trip_b = jnp.where(
        (i % 2) == 0,
        jnp.maximum(0, (jnp.minimum(31, 63 - i) - 15) // 2), 0)

    def bbody(k, M):
        j0 = i + 16 + 2 * k
        Mn = _absorb(list(M), xbuf, j0)
        _store_row(rowbuf, j0, zeros)
        Mn = _absorb(Mn, xbuf, j0 + 1)
        _store_row(rowbuf, j0 + 1, Mn)
        return tuple(Mn)
    M = lax.fori_loop(0, trip_b, bbody, M)

    trip_c = jnp.where((i % 4) == 0, jnp.maximum(0, (28 - i) // 4 + 1), 0)

    def cbody(k, M):
        j0 = i + 32 + 4 * k
        Mn = list(M)
        for o in range(3):
            Mn = _absorb(Mn, xbuf, j0 + o)
            _store_row(rowbuf, j0 + o, zeros)
        Mn = _absorb(Mn, xbuf, j0 + 3)
        _store_row(rowbuf, j0 + 3, Mn)
        return tuple(Mn)
    M = lax.fori_loop(0, trip_c, cbody, M)

    p_tail = i + trip_a + 1 + 2 * trip_b + 4 * trip_c

    def ztail(jj, c):
        _store_row(rowbuf, jj, zeros)
        return c
    lax.fori_loop(p_tail, N, ztail, 0)


def _sc_band_max(x_hbm, out_hbm, xbuf, row0, row1, sem0, sem1):
    wid = lax.axis_index("s") * 2 + lax.axis_index("c")
    b = wid // 2
    ibase = (wid % 2) * (N // 2)
    pltpu.sync_copy(x_hbm.at[b], xbuf)

    lane = lax.iota(jnp.int32, 16)
    zero = (lane & 0).astype(jnp.float32)
    zeros = [zero] * NL

    bufs = ((row0, sem0), (row1, sem1))

    def ring_body(rr, carry):
        for phase, (buf, sem) in enumerate(bufs):
            i = ibase + 2 * rr + phase
            cp = pltpu.make_async_copy(buf, out_hbm.at[b, i], sem)

            @pl.when(rr > 0)
            def _():
                cp.wait()

            _do_row(i, xbuf, buf, zeros)
            cp.start()
        return carry

    lax.fori_loop(0, N // 4, ring_body, 0)
    for phase, (buf, sem) in enumerate(bufs):
        pltpu.make_async_copy(
            buf, out_hbm.at[b, ibase + N // 2 - 2 + phase], sem).wait()


@jax.jit
def _run(xt):
    mesh = plsc.VectorSubcoreMesh(core_axis_name="c", subcore_axis_name="s")
    f = pl.kernel(
        _sc_band_max,
        mesh=mesh,
        out_type=jax.ShapeDtypeStruct((B, N, N, CH), jnp.float32),
        scratch_types=[
            pltpu.VMEM((N, CH), jnp.float32),
            pltpu.VMEM((N, CH), jnp.float32),
            pltpu.VMEM((N, CH), jnp.float32),
            pltpu.SemaphoreType.DMA,
            pltpu.SemaphoreType.DMA,
        ],
    )
    return f(xt)


def kernel(x):
    xt = jnp.transpose(x, (0, 2, 1))
    out = _run(xt)
    return jnp.transpose(out, (0, 3, 1, 2))


# speedup vs baseline: 3.2143x; 1.0033x over previous
import functools

import numpy as np
import jax
import jax.numpy as jnp
from jax import lax
from jax.experimental import pallas as pl
from jax.experimental.pallas import tpu as pltpu
from jax.experimental.pallas import tpu_sc as plsc

N = 64
CH = 512
NL = CH // 16
B = 16


def _load_row(xbuf, j):
    return [xbuf[j, pl.ds(k * 16, 16)] for k in range(NL)]


def _absorb(M, xbuf, j):
    return [jnp.maximum(m, v) for m, v in zip(M, _load_row(xbuf, j))]


def _store_row(buf, j, vals):
    for k in range(NL):
        buf[j, pl.ds(k * 16, 16)] = vals[k]


def _do_row(i, xbuf, rowbuf, zeros):
    def zpre(jj, c):
        _store_row(rowbuf, jj, zeros)
        return c
    lax.fori_loop(0, i, zpre, 0)

    M = tuple(_load_row(xbuf, i))
    _store_row(rowbuf, i, M)

    trip_a = jnp.minimum(15, 63 - i)

    def abody(k, M):
        j = i + k
        Mn = _absorb(list(M), xbuf, j)
        _store_row(rowbuf, j, Mn)
        return tuple(Mn)
    M = lax.fori_loop(1, trip_a + 1, abody, M)

    trip_b = jnp.where(
        (i % 2) == 0,
        jnp.maximum(0, (jnp.minimum(31, 63 - i) - 15) // 2), 0)

    def bbody(k, M):
        j0 = i + 16 + 2 * k
        Mn = _absorb(list(M), xbuf, j0)
        _store_row(rowbuf, j0, zeros)
        Mn = _absorb(Mn, xbuf, j0 + 1)
        _store_row(rowbuf, j0 + 1, Mn)
        return tuple(Mn)
    M = lax.fori_loop(0, trip_b, bbody, M)

    trip_c = jnp.where((i % 4) == 0, jnp.maximum(0, (28 - i) // 4 + 1), 0)

    def cbody(k, M):
        j0 = i + 32 + 4 * k
        Mn = list(M)
        for o in range(3):
            Mn = _absorb(Mn, xbuf, j0 + o)
            _store_row(rowbuf, j0 + o, zeros)
        Mn = _absorb(Mn, xbuf, j0 + 3)
        _store_row(rowbuf, j0 + 3, Mn)
        return tuple(Mn)
    M = lax.fori_loop(0, trip_c, cbody, M)

    p_tail = i + trip_a + 1 + 2 * trip_b + 4 * trip_c

    def ztail(jj, c):
        _store_row(rowbuf, jj, zeros)
        return c
    lax.fori_loop(p_tail, N, ztail, 0)


def _sc_band_max(x_hbm, out_hbm, xbuf, row0, row1, sem0, sem1):
    wid = lax.axis_index("s") * 2 + lax.axis_index("c")
    b = wid // 2
    ibase = ((b + wid) % 2) * (N // 2)
    pltpu.sync_copy(x_hbm.at[b], xbuf)

    lane = lax.iota(jnp.int32, 16)
    zero = (lane & 0).astype(jnp.float32)
    zeros = [zero] * NL

    bufs = ((row0, sem0), (row1, sem1))

    def ring_body(rr, carry):
        for phase, (buf, sem) in enumerate(bufs):
            i = ibase + 2 * rr + phase
            cp = pltpu.make_async_copy(buf, out_hbm.at[b, i], sem)

            @pl.when(rr > 0)
            def _():
                cp.wait()

            _do_row(i, xbuf, buf, zeros)
            cp.start()
        return carry

    lax.fori_loop(0, N // 4, ring_body, 0)
    for phase, (buf, sem) in enumerate(bufs):
        pltpu.make_async_copy(
            buf, out_hbm.at[b, ibase + N // 2 - 2 + phase], sem).wait()


@jax.jit
def _run(xt):
    mesh = plsc.VectorSubcoreMesh(core_axis_name="c", subcore_axis_name="s")
    f = pl.kernel(
        _sc_band_max,
        mesh=mesh,
        out_type=jax.ShapeDtypeStruct((B, N, N, CH), jnp.float32),
        scratch_types=[
            pltpu.VMEM((N, CH), jnp.float32),
            pltpu.VMEM((N, CH), jnp.float32),
            pltpu.VMEM((N, CH), jnp.float32),
            pltpu.SemaphoreType.DMA,
            pltpu.SemaphoreType.DMA,
        ],
    )
    return f(xt)


def kernel(x):
    xt = jnp.transpose(x, (0, 2, 1))
    out = _run(xt)
    return jnp.transpose(out, (0, 3, 1, 2))
